# baseline (speedup 1.0000x reference)
import functools
import jax
import jax.numpy as jnp
from jax import lax
from jax.experimental import pallas as pl
from jax.experimental.pallas import tpu as pltpu
from jax.experimental.pallas import tpu_sc as plsc

_N = 100000
_E = 400000
_B = 5000
_H = 32
_NP = 100352
_EP = 401408
_BP = 5120
_NBLK = 1024
_EBLK = 2048

_VMESH = plsc.VectorSubcoreMesh(core_axis_name="c", subcore_axis_name="s")
_SC_PARAMS = pltpu.CompilerParams(use_tc_tiling_on_sc=False)
_EC = 128
_ECH = _EP // _EC
_CH1 = _ECH // 16
_CH2 = _CH1 // 2
_BC = 14
_EB = _BC * _EC
_NB1 = _CH1 // _BC
_NB2 = _CH2 // _BC
_NPT = _NP // 16
_NPW = _NP // 32
_BPT = _BP // 16
_NCH = _NP // _EC
_PCH1 = _NCH // 16


def _elu(v):
    return jnp.where(v > 0, v, jnp.exp(jnp.minimum(v, 0.0)) - 1.0)


def _f32(shape):
    return jax.ShapeDtypeStruct(shape, jnp.float32)


def _k_node0(x_ref, W0_ref, b0_ref, Wl_ref, as_ref, ad_ref,
             h_ref, hWa_ref, hWb_ref, hs_ref, hd_ref, mhs_ref, mhd_ref):
    i = pl.program_id(0)
    h = _elu(jnp.dot(x_ref[...], W0_ref[...],
                     preferred_element_type=jnp.float32) + b0_ref[...][None, :])
    h_ref[...] = h
    hW = jnp.dot(h, Wl_ref[...], preferred_element_type=jnp.float32)
    hWa_ref[...] = hW[:, :16]
    hWb_ref[...] = hW[:, 16:]
    hs = jnp.dot(hW, as_ref[...], preferred_element_type=jnp.float32)
    hd = jnp.dot(hW, ad_ref[...], preferred_element_type=jnp.float32)
    hs_ref[...] = hs
    hd_ref[...] = hd

    @pl.when(i == 0)
    def _():
        mhs_ref[...] = jnp.full((1, 1), -jnp.inf)
        mhd_ref[...] = jnp.full((1, 1), -jnp.inf)

    mhs_ref[...] = jnp.maximum(mhs_ref[...], jnp.max(hs))
    mhd_ref[...] = jnp.maximum(mhd_ref[...], jnp.max(hd))


def _k_node(aggA_ref, aggB_ref, den_ref, hp_ref, Wl_ref, as_ref, ad_ref,
            h_ref, hWa_ref, hWb_ref, hs_ref, hd_ref, mhs_ref, mhd_ref):
    i = pl.program_id(0)
    inv = 1.0 / (den_ref[...] + 1e-9)
    agg = jnp.concatenate(
        [aggA_ref[0] + aggA_ref[1], aggB_ref[0] + aggB_ref[1]],
        axis=1) * inv[:, None]
    h = _elu(agg) + hp_ref[...]
    h_ref[...] = h
    hW = jnp.dot(h, Wl_ref[...], preferred_element_type=jnp.float32)
    hWa_ref[...] = hW[:, :16]
    hWb_ref[...] = hW[:, 16:]
    hs = jnp.dot(hW, as_ref[...], preferred_element_type=jnp.float32)
    hd = jnp.dot(hW, ad_ref[...], preferred_element_type=jnp.float32)
    hs_ref[...] = hs
    hd_ref[...] = hd

    @pl.when(i == 0)
    def _():
        mhs_ref[...] = jnp.full((1, 1), -jnp.inf)
        mhd_ref[...] = jnp.full((1, 1), -jnp.inf)

    mhs_ref[...] = jnp.maximum(mhs_ref[...], jnp.max(hs))
    mhd_ref[...] = jnp.maximum(mhd_ref[...], jnp.max(hd))


def _k_final(aggA_ref, aggB_ref, den_ref, hp_ref, wp_ref,
             h_ref, s_ref, ms_ref):
    i = pl.program_id(0)
    inv = 1.0 / (den_ref[...] + 1e-9)
    agg = jnp.concatenate(
        [aggA_ref[0] + aggA_ref[1], aggB_ref[0] + aggB_ref[1]],
        axis=1) * inv[:, None]
    h = _elu(agg) + hp_ref[...]
    h_ref[...] = h
    s = jnp.dot(h, wp_ref[...], preferred_element_type=jnp.float32)
    s_ref[...] = s

    @pl.when(i == 0)
    def _():
        ms_ref[...] = jnp.full((1, 1), -jnp.inf)

    ms_ref[...] = jnp.maximum(ms_ref[...], jnp.max(s))


def _k_edge(eattr_ref, Wel_ref, ae_ref, eWa_ref, eWb_ref, ea_ref, mea_ref):
    j = pl.program_id(0)
    ms = []
    for l in range(3):
        eW = jnp.dot(eattr_ref[...], Wel_ref[l],
                     preferred_element_type=jnp.float32)
        eWa_ref[l] = eW[:, :16]
        eWb_ref[l] = eW[:, 16:]
        ea = jnp.dot(eW, ae_ref[l], preferred_element_type=jnp.float32)
        ea_ref[l] = ea
        ms.append(jnp.max(ea))

    @pl.when(j == 0)
    def _():
        mea_ref[...] = jnp.full((3, 1), -jnp.inf)

    mea_ref[...] = jnp.maximum(mea_ref[...], jnp.stack(ms)[:, None])


def _k_mlp(pooled_ref, wd_ref, t_ref, don_ref, acc_ref,
           W1_ref, b1_ref, W2_ref, b2_ref, W3_ref, b3_ref, out_ref):
    wd = wd_ref[0, :_B] + wd_ref[1, :_B]
    pooled = (pooled_ref[0, :_B, :] + pooled_ref[1, :_B, :]) \
        / (wd + 1e-9)[:, None]
    W1 = W1_ref[...]
    z = (jnp.dot(pooled, W1[:_H], preferred_element_type=jnp.float32)
         + t_ref[...][:, None] * W1[_H][None, :]
         + don_ref[...][:, None] * W1[_H + 1][None, :]
         + acc_ref[...][:, None] * W1[_H + 2][None, :]
         + b1_ref[...][None, :])
    z = _elu(z)
    z = _elu(jnp.dot(z, W2_ref[...], preferred_element_type=jnp.float32)
             + b2_ref[...][None, :])
    logp = (jnp.dot(z, W3_ref[...], preferred_element_type=jnp.float32)
            + b3_ref[...][None, :])
    out_ref[...] = jnp.exp(logp[:, 0])


_NODE0 = pl.pallas_call(
    _k_node0,
    grid=(_NP // _NBLK,),
    in_specs=[
        pl.BlockSpec((_NBLK, 24), lambda i: (i, 0)),
        pl.BlockSpec((24, _H), lambda i: (0, 0)),
        pl.BlockSpec((_H,), lambda i: (0,)),
        pl.BlockSpec((_H, _H), lambda i: (0, 0)),
        pl.BlockSpec((_H,), lambda i: (0,)),
        pl.BlockSpec((_H,), lambda i: (0,)),
    ],
    out_specs=[
        pl.BlockSpec((_NBLK, _H), lambda i: (i, 0)),
        pl.BlockSpec((_NBLK, 16), lambda i: (i, 0)),
        pl.BlockSpec((_NBLK, 16), lambda i: (i, 0)),
        pl.BlockSpec((_NBLK,), lambda i: (i,)),
        pl.BlockSpec((_NBLK,), lambda i: (i,)),
        pl.BlockSpec((1, 1), lambda i: (0, 0)),
        pl.BlockSpec((1, 1), lambda i: (0, 0)),
    ],
    out_shape=[_f32((_NP, _H)), _f32((_NP, 16)), _f32((_NP, 16)),
               _f32((_NP,)), _f32((_NP,)), _f32((1, 1)), _f32((1, 1))],
)

_NODE = pl.pallas_call(
    _k_node,
    grid=(_NP // _NBLK,),
    in_specs=[
        pl.BlockSpec((2, _NBLK, 16), lambda i: (0, i, 0)),
        pl.BlockSpec((2, _NBLK, 16), lambda i: (0, i, 0)),
        pl.BlockSpec((_NBLK,), lambda i: (i,)),
        pl.BlockSpec((_NBLK, _H), lambda i: (i, 0)),
        pl.BlockSpec((_H, _H), lambda i: (0, 0)),
        pl.BlockSpec((_H,), lambda i: (0,)),
        pl.BlockSpec((_H,), lambda i: (0,)),
    ],
    out_specs=[
        pl.BlockSpec((_NBLK, _H), lambda i: (i, 0)),
        pl.BlockSpec((_NBLK, 16), lambda i: (i, 0)),
        pl.BlockSpec((_NBLK, 16), lambda i: (i, 0)),
        pl.BlockSpec((_NBLK,), lambda i: (i,)),
        pl.BlockSpec((_NBLK,), lambda i: (i,)),
        pl.BlockSpec((1, 1), lambda i: (0, 0)),
        pl.BlockSpec((1, 1), lambda i: (0, 0)),
    ],
    out_shape=[_f32((_NP, _H)), _f32((_NP, 16)), _f32((_NP, 16)),
               _f32((_NP,)), _f32((_NP,)), _f32((1, 1)), _f32((1, 1))],
)

_FINAL = pl.pallas_call(
    _k_final,
    grid=(_NP // _NBLK,),
    in_specs=[
        pl.BlockSpec((2, _NBLK, 16), lambda i: (0, i, 0)),
        pl.BlockSpec((2, _NBLK, 16), lambda i: (0, i, 0)),
        pl.BlockSpec((_NBLK,), lambda i: (i,)),
        pl.BlockSpec((_NBLK, _H), lambda i: (i, 0)),
        pl.BlockSpec((_H,), lambda i: (0,)),
    ],
    out_specs=[
        pl.BlockSpec((_NBLK, _H), lambda i: (i, 0)),
        pl.BlockSpec((_NBLK,), lambda i: (i,)),
        pl.BlockSpec((1, 1), lambda i: (0, 0)),
    ],
    out_shape=[_f32((_NP, _H)), _f32((_NP,)), _f32((1, 1))],
)

_EDGE = pl.pallas_call(
    _k_edge,
    grid=(_EP // _EBLK,),
    in_specs=[
        pl.BlockSpec((_EBLK, 9), lambda j: (j, 0)),
        pl.BlockSpec((3, 9, _H), lambda j: (0, 0, 0)),
        pl.BlockSpec((3, _H), lambda j: (0, 0)),
    ],
    out_specs=[
        pl.BlockSpec((3, _EBLK, 16), lambda j: (0, j, 0)),
        pl.BlockSpec((3, _EBLK, 16), lambda j: (0, j, 0)),
        pl.BlockSpec((3, _EBLK), lambda j: (0, j)),
        pl.BlockSpec((3, 1), lambda j: (0, 0)),
    ],
    out_shape=[_f32((3, _EP, 16)), _f32((3, _EP, 16)), _f32((3, _EP)),
               _f32((3, 1))],
)

_MLP = pl.pallas_call(
    _k_mlp,
    grid=(1,),
    in_specs=[
        pl.BlockSpec((2, _BP, _H), lambda i: (0, 0, 0)),
        pl.BlockSpec((2, _BP), lambda i: (0, 0)),
        pl.BlockSpec((_B,), lambda i: (0,)),
        pl.BlockSpec((_B,), lambda i: (0,)),
        pl.BlockSpec((_B,), lambda i: (0,)),
        pl.BlockSpec((_H + 3, _H), lambda i: (0, 0)),
        pl.BlockSpec((_H,), lambda i: (0,)),
        pl.BlockSpec((_H, 16), lambda i: (0, 0)),
        pl.BlockSpec((16,), lambda i: (0,)),
        pl.BlockSpec((16, 1), lambda i: (0, 0)),
        pl.BlockSpec((1,), lambda i: (0,)),
    ],
    out_specs=pl.BlockSpec((_B,), lambda i: (0,)),
    out_shape=_f32((_B,)),
)


def _sck1_body(l, src2, dst2, ea_h, cap_h, hs_h, hd_h, zn1_h,
               w_h, den_h,
               sidxb, didxb, eavb, hsgb, hdgb, wcbb, capv, denom_sh, sem):
    t = lax.axis_index("s")
    c = lax.axis_index("c")
    pltpu.sync_copy(cap_h, capv)
    pltpu.sync_copy(zn1_h, denom_sh.at[pl.ds(t * _NPT, _NPT)])
    plsc.subcore_barrier()
    cap = capv[...]

    @pl.loop(0, _NB1)
    def _(b):
        g0 = t * _CH1 + b * _BC
        e0 = g0 * _EC
        d1 = pltpu.async_copy(src2.at[pl.ds(g0, _BC), :], sidxb, sem)
        d2 = pltpu.async_copy(dst2.at[pl.ds(g0, _BC), :], didxb, sem)
        d3 = pltpu.async_copy(ea_h.at[l, pl.ds(e0, _EB)], eavb, sem)
        d1.wait()
        d2.wait()
        d3.wait()
        gs = []
        for j in range(_BC):
            gs.append(pltpu.async_copy(
                hs_h.at[sidxb.at[j]], hsgb.at[pl.ds(j * _EC, _EC)], sem))
            gs.append(pltpu.async_copy(
                hd_h.at[didxb.at[j]], hdgb.at[pl.ds(j * _EC, _EC)], sem))
        for g in gs:
            g.wait()
        for k in range(_EB // 16):
            sl = pl.ds(k * 16, 16)
            lg = hsgb[sl] + hdgb[sl] + eavb[sl]
            lg = jnp.where(lg > 0, lg, 0.2 * lg)
            wcbb[sl] = jnp.exp(lg - cap)

        own = (b >= c * _NB2) & (b < (c + 1) * _NB2)

        @pl.when(own)
        def _():
            pltpu.sync_copy(wcbb, w_h.at[pl.ds(e0, _EB)])

        ss = []
        for j in range(_BC):
            ss.append(pltpu.async_copy(
                wcbb.at[pl.ds(j * _EC, _EC)], denom_sh.at[didxb.at[j]],
                sem, add=True))
        for s in ss:
            s.wait()

    plsc.subcore_barrier()
    w0 = (2 * t + c) * _NPW
    pltpu.sync_copy(denom_sh.at[pl.ds(w0, _NPW)], den_h.at[pl.ds(w0, _NPW)])


def _sck1(l, src2, dst2, ea, cap16, hs, hd, zn1):
    f = pl.kernel(
        functools.partial(_sck1_body, l),
        out_type=[_f32((_EP,)), _f32((_NP,))],
        mesh=_VMESH,
        compiler_params=_SC_PARAMS,
        scratch_types=[
            pltpu.VMEM((_BC, _EC), jnp.int32),
            pltpu.VMEM((_BC, _EC), jnp.int32),
            pltpu.VMEM((_EB,), jnp.float32),
            pltpu.VMEM((_EB,), jnp.float32),
            pltpu.VMEM((_EB,), jnp.float32),
            pltpu.VMEM((_EB,), jnp.float32),
            pltpu.VMEM((16,), jnp.float32),
            pltpu.VMEM_SHARED((_NP,), jnp.float32),
            pltpu.SemaphoreType.DMA,
        ],
    )
    return f(src2, dst2, ea, cap16, hs, hd, zn1)


def _sck2_body(l, src2, dst2, w_h, hWa_h, hWb_h, eWa_h, eWb_h, zn16_h,
               aggA_h, aggB_h,
               sidx, didx, av, rows, erows, agg_sh, sem):
    t = lax.axis_index("s")
    c = lax.axis_index("c")
    pltpu.sync_copy(zn16_h, agg_sh.at[pl.ds(t * _NPT, _NPT), :])
    plsc.subcore_barrier()

    def half(hW_h, eW_h, agg_out):
        @pl.loop(0, _CH2)
        def _(i):
            gi = t * _CH1 + c * _CH2 + i
            e0 = gi * _EC
            d1 = pltpu.async_copy(src2.at[gi], sidx, sem)
            d2 = pltpu.async_copy(dst2.at[gi], didx, sem)
            d3 = pltpu.async_copy(w_h.at[pl.ds(e0, _EC)],
                                  av.at[pl.ds(0, _EC)], sem)
            d4 = pltpu.async_copy(eW_h.at[l, pl.ds(e0, _EC), :], erows, sem)
            d1.wait()
            g = pltpu.async_copy(hW_h.at[sidx], rows, sem)
            d2.wait()
            d3.wait()
            d4.wait()
            g.wait()

            @pl.loop(0, _EC)
            def _(r):
                a = av[pl.ds(r, 16)][0]
                rows[r] = (rows[r] + erows[r]) * a

            pltpu.sync_copy(rows, agg_sh.at[didx], add=True)

        plsc.subcore_barrier()
        pltpu.sync_copy(agg_sh.at[pl.ds(t * _NPT, _NPT), :],
                        agg_out.at[c, pl.ds(t * _NPT, _NPT), :])

    half(hWa_h, eWa_h, aggA_h)
    pltpu.sync_copy(zn16_h, agg_sh.at[pl.ds(t * _NPT, _NPT), :])
    plsc.subcore_barrier()
    half(hWb_h, eWb_h, aggB_h)


def _sck2(l, src2, dst2, w, hWa, hWb, eWa, eWb, zn16):
    f = pl.kernel(
        functools.partial(_sck2_body, l),
        out_type=[_f32((2, _NP, 16)), _f32((2, _NP, 16))],
        mesh=_VMESH,
        compiler_params=_SC_PARAMS,
        scratch_types=[
            pltpu.VMEM((_EC,), jnp.int32),
            pltpu.VMEM((_EC,), jnp.int32),
            pltpu.VMEM((_EC + 16,), jnp.float32),
            pltpu.VMEM((_EC, 16), jnp.float32),
            pltpu.VMEM((_EC, 16), jnp.float32),
            pltpu.VMEM_SHARED((_NP, 16), jnp.float32),
            pltpu.SemaphoreType.DMA,
        ],
    )
    return f(src2, dst2, w, hWa, hWb, eWa, eWb, zn16)


def _sc_pool_body(batch2, s2, h3, cap_h, zb1_h, zb16_h,
                  pooled_h, wd_h,
                  bidxb, svb, wsbb, hrowb, capv, wd_sh, pool_sh, sem):
    t = lax.axis_index("s")
    c = lax.axis_index("c")
    pltpu.sync_copy(cap_h, capv)
    pltpu.sync_copy(zb1_h, wd_sh.at[pl.ds(t * _BPT, _BPT)])
    pltpu.sync_copy(zb16_h, pool_sh.at[pl.ds(t * _BPT, _BPT), :, :])
    plsc.subcore_barrier()
    cap = capv[...]

    @pl.loop(0, 7)
    def _(b):
        g0 = t * _PCH1 + b * 7
        n0 = g0 * _EC
        d1 = pltpu.async_copy(batch2.at[pl.ds(g0, 7), :], bidxb, sem)
        d2 = pltpu.async_copy(s2.at[pl.ds(g0, 7), :], svb, sem)
        d3 = pltpu.async_copy(h3.at[pl.ds(n0, 7 * _EC), :, :], hrowb, sem)
        d1.wait()
        d2.wait()
        d3.wait()
        for j in range(7):
            for k in range(8):
                wsbb[pl.ds((j * 8 + k) * 16, 16)] = jnp.exp(
                    svb[j, pl.ds(k * 16, 16)] - cap)
        ws_sc = []
        for j in range(7):
            ws_sc.append(pltpu.async_copy(
                wsbb.at[pl.ds(j * _EC, _EC)], wd_sh.at[bidxb.at[j]],
                sem, add=True))

        @pl.loop(0, 7 * _EC)
        def _(r):
            a = wsbb[pl.ds(r, 16)][0]
            hrowb[r, 0] = hrowb[r, 0] * a
            hrowb[r, 1] = hrowb[r, 1] * a

        for s in ws_sc:
            s.wait()
        ps = []
        for j in range(7):
            ps.append(pltpu.async_copy(
                hrowb.at[pl.ds(j * _EC, _EC), :, :],
                pool_sh.at[bidxb.at[j]], sem, add=True))
        for s in ps:
            s.wait()

    plsc.subcore_barrier()
    pltpu.sync_copy(pool_sh.at[pl.ds(t * _BPT, _BPT), :, :],
                    pooled_h.at[c, pl.ds(t * _BPT, _BPT), :, :])
    pltpu.sync_copy(wd_sh.at[pl.ds(t * _BPT, _BPT)],
                    wd_h.at[c, pl.ds(t * _BPT, _BPT)])


def _sc_pool(batch2, s2, h3, cap16, zb1, zb16):
    f = pl.kernel(
        _sc_pool_body,
        out_type=[_f32((2, _BP, 2, 16)), _f32((2, _BP))],
        mesh=_VMESH,
        compiler_params=_SC_PARAMS,
        scratch_types=[
            pltpu.VMEM((7, _EC), jnp.int32),
            pltpu.VMEM((7, _EC), jnp.float32),
            pltpu.VMEM((7 * _EC + 16,), jnp.float32),
            pltpu.VMEM((7 * _EC, 2, 16), jnp.float32),
            pltpu.VMEM((16,), jnp.float32),
            pltpu.VMEM_SHARED((_BP,), jnp.float32),
            pltpu.VMEM_SHARED((_BP, 2, 16), jnp.float32),
            pltpu.SemaphoreType.DMA,
        ],
    )
    return f(batch2, s2, h3, cap16, zb1, zb16)


def kernel(x, temperature, edge_index, edge_attr, numHDonors, numHAcceptors,
           batch, W0, b0, Wl, Wel, a_s, a_d, a_e, w_pool, W1, b1, W2, b2,
           W3, b3):
    x_p = jnp.zeros((_NP, 24), jnp.float32).at[:_N].set(x)
    src = jnp.zeros((_EP,), jnp.int32).at[:_E].set(edge_index[0])
    dst = jnp.full((_EP,), _N, jnp.int32).at[:_E].set(edge_index[1])
    eattr_p = jnp.zeros((_EP, 9), jnp.float32).at[:_E].set(edge_attr)
    batch_p = jnp.full((_NP,), _B, jnp.int32).at[:_N].set(batch)
    src2 = src.reshape(_ECH, _EC)
    dst2 = dst.reshape(_ECH, _EC)
    batch2 = batch_p.reshape(_NCH, _EC)
    zn16 = jnp.zeros((_NPT, 16), jnp.float32)
    zn1 = jnp.zeros((_NPT,), jnp.float32)
    zb1 = jnp.zeros((_BPT,), jnp.float32)
    zb16 = jnp.zeros((_BPT, 2, 16), jnp.float32)

    h, hWa, hWb, hs, hd, mhs, mhd = _NODE0(x_p, W0, b0, Wl[0], a_s[0], a_d[0])
    eWa3, eWb3, ea3, mea3 = _EDGE(eattr_p, Wel, a_e)
    for l in range(3):
        cap = jnp.maximum(mhs[0, 0] + mhd[0, 0] + mea3[l, 0], 0.0)
        cap16 = jnp.full((16,), cap, jnp.float32)
        w, den = _sck1(l, src2, dst2, ea3, cap16, hs, hd, zn1)
        aggA, aggB = _sck2(l, src2, dst2, w, hWa, hWb, eWa3, eWb3, zn16)
        if l < 2:
            h, hWa, hWb, hs, hd, mhs, mhd = _NODE(
                aggA, aggB, den, h, Wl[l + 1], a_s[l + 1], a_d[l + 1])
        else:
            h, s, ms = _FINAL(aggA, aggB, den, h, w_pool)
    cap16s = jnp.full((16,), ms[0, 0], jnp.float32)
    pooled4, wd2 = _sc_pool(batch2, s.reshape(_NCH, _EC),
                            h.reshape(_NP, 2, 16), cap16s, zb1, zb16)
    return _MLP(pooled4.reshape(2, _BP, _H), wd2, temperature,
                numHDonors.astype(jnp.float32),
                numHAcceptors.astype(jnp.float32), W1, b1, W2, b2, W3, b3)


# speedup vs baseline: 1.0765x; 1.0765x over previous
import functools
import jax
import jax.numpy as jnp
from jax import lax
from jax.experimental import pallas as pl
from jax.experimental.pallas import tpu as pltpu
from jax.experimental.pallas import tpu_sc as plsc

_N = 100000
_E = 400000
_B = 5000
_H = 32
_NP = 100352
_EP = 401408
_BP = 5120
_NBLK = 1024
_EBLK = 2048

_VMESH = plsc.VectorSubcoreMesh(core_axis_name="c", subcore_axis_name="s")
_SC_PARAMS = pltpu.CompilerParams(use_tc_tiling_on_sc=False)
_EC = 128
_ECH = _EP // _EC
_CH1 = _ECH // 16
_CH2 = _CH1 // 2
_BC = 14
_EB = _BC * _EC
_NB1 = _CH1 // _BC
_NB2 = _CH2 // _BC
_NPT = _NP // 16
_NPW = _NP // 32
_BPT = _BP // 16
_NCH = _NP // _EC
_PCH1 = _NCH // 16


def _elu(v):
    return jnp.where(v > 0, v, jnp.exp(jnp.minimum(v, 0.0)) - 1.0)


def _f32(shape):
    return jax.ShapeDtypeStruct(shape, jnp.float32)


def _k_node0(x_ref, W0_ref, b0_ref, Wl_ref, as_ref, ad_ref,
             h_ref, hWa_ref, hWb_ref, hs_ref, hd_ref, mhs_ref, mhd_ref):
    i = pl.program_id(0)
    h = _elu(jnp.dot(x_ref[...], W0_ref[...],
                     preferred_element_type=jnp.float32) + b0_ref[...][None, :])
    h_ref[...] = h
    hW = jnp.dot(h, Wl_ref[...], preferred_element_type=jnp.float32)
    hWa_ref[...] = hW[:, :16]
    hWb_ref[...] = hW[:, 16:]
    hs = jnp.dot(hW, as_ref[...], preferred_element_type=jnp.float32)
    hd = jnp.dot(hW, ad_ref[...], preferred_element_type=jnp.float32)
    hs_ref[...] = hs
    hd_ref[...] = hd

    @pl.when(i == 0)
    def _():
        mhs_ref[...] = jnp.full((1, 1), -jnp.inf)
        mhd_ref[...] = jnp.full((1, 1), -jnp.inf)

    mhs_ref[...] = jnp.maximum(mhs_ref[...], jnp.max(hs))
    mhd_ref[...] = jnp.maximum(mhd_ref[...], jnp.max(hd))


def _k_node(aggA_ref, aggB_ref, den_ref, hp_ref, Wl_ref, as_ref, ad_ref,
            h_ref, hWa_ref, hWb_ref, hs_ref, hd_ref, mhs_ref, mhd_ref):
    i = pl.program_id(0)
    inv = 1.0 / (den_ref[...] + 1e-9)
    agg = jnp.concatenate(
        [aggA_ref[0] + aggA_ref[1], aggB_ref[0] + aggB_ref[1]],
        axis=1) * inv[:, None]
    h = _elu(agg) + hp_ref[...]
    h_ref[...] = h
    hW = jnp.dot(h, Wl_ref[...], preferred_element_type=jnp.float32)
    hWa_ref[...] = hW[:, :16]
    hWb_ref[...] = hW[:, 16:]
    hs = jnp.dot(hW, as_ref[...], preferred_element_type=jnp.float32)
    hd = jnp.dot(hW, ad_ref[...], preferred_element_type=jnp.float32)
    hs_ref[...] = hs
    hd_ref[...] = hd

    @pl.when(i == 0)
    def _():
        mhs_ref[...] = jnp.full((1, 1), -jnp.inf)
        mhd_ref[...] = jnp.full((1, 1), -jnp.inf)

    mhs_ref[...] = jnp.maximum(mhs_ref[...], jnp.max(hs))
    mhd_ref[...] = jnp.maximum(mhd_ref[...], jnp.max(hd))


def _k_final(aggA_ref, aggB_ref, den_ref, hp_ref, wp_ref,
             h_ref, s_ref, ms_ref):
    i = pl.program_id(0)
    inv = 1.0 / (den_ref[...] + 1e-9)
    agg = jnp.concatenate(
        [aggA_ref[0] + aggA_ref[1], aggB_ref[0] + aggB_ref[1]],
        axis=1) * inv[:, None]
    h = _elu(agg) + hp_ref[...]
    h_ref[...] = h
    s = jnp.dot(h, wp_ref[...], preferred_element_type=jnp.float32)
    s_ref[...] = s

    @pl.when(i == 0)
    def _():
        ms_ref[...] = jnp.full((1, 1), -jnp.inf)

    ms_ref[...] = jnp.maximum(ms_ref[...], jnp.max(s))


def _k_edge(eattr_ref, Wel_ref, ae_ref, eWa_ref, eWb_ref, ea_ref, mea_ref):
    j = pl.program_id(0)
    eW = jnp.dot(eattr_ref[...], Wel_ref[0],
                 preferred_element_type=jnp.float32)
    eWa_ref[...] = eW[None, :, :16]
    eWb_ref[...] = eW[None, :, 16:]
    ea = jnp.dot(eW, ae_ref[0], preferred_element_type=jnp.float32)
    ea_ref[0] = ea

    @pl.when(j == 0)
    def _():
        mea_ref[...] = jnp.full((1, 1), -jnp.inf)

    mea_ref[...] = jnp.maximum(mea_ref[...], jnp.max(ea))


def _k_mlp(pooled_ref, wd_ref, t_ref, don_ref, acc_ref,
           W1_ref, b1_ref, W2_ref, b2_ref, W3_ref, b3_ref, out_ref):
    wd = wd_ref[0, :_B] + wd_ref[1, :_B]
    pooled = (pooled_ref[0, :_B, :] + pooled_ref[1, :_B, :]) \
        / (wd + 1e-9)[:, None]
    W1 = W1_ref[...]
    z = (jnp.dot(pooled, W1[:_H], preferred_element_type=jnp.float32)
         + t_ref[...][:, None] * W1[_H][None, :]
         + don_ref[...][:, None] * W1[_H + 1][None, :]
         + acc_ref[...][:, None] * W1[_H + 2][None, :]
         + b1_ref[...][None, :])
    z = _elu(z)
    z = _elu(jnp.dot(z, W2_ref[...], preferred_element_type=jnp.float32)
             + b2_ref[...][None, :])
    logp = (jnp.dot(z, W3_ref[...], preferred_element_type=jnp.float32)
            + b3_ref[...][None, :])
    out_ref[...] = jnp.exp(logp[:, 0])


_NODE0 = pl.pallas_call(
    _k_node0,
    grid=(_NP // _NBLK,),
    in_specs=[
        pl.BlockSpec((_NBLK, 24), lambda i: (i, 0)),
        pl.BlockSpec((24, _H), lambda i: (0, 0)),
        pl.BlockSpec((_H,), lambda i: (0,)),
        pl.BlockSpec((_H, _H), lambda i: (0, 0)),
        pl.BlockSpec((_H,), lambda i: (0,)),
        pl.BlockSpec((_H,), lambda i: (0,)),
    ],
    out_specs=[
        pl.BlockSpec((_NBLK, _H), lambda i: (i, 0)),
        pl.BlockSpec((_NBLK, 16), lambda i: (i, 0)),
        pl.BlockSpec((_NBLK, 16), lambda i: (i, 0)),
        pl.BlockSpec((_NBLK,), lambda i: (i,)),
        pl.BlockSpec((_NBLK,), lambda i: (i,)),
        pl.BlockSpec((1, 1), lambda i: (0, 0)),
        pl.BlockSpec((1, 1), lambda i: (0, 0)),
    ],
    out_shape=[_f32((_NP, _H)), _f32((_NP, 16)), _f32((_NP, 16)),
               _f32((_NP,)), _f32((_NP,)), _f32((1, 1)), _f32((1, 1))],
)

_NODE = pl.pallas_call(
    _k_node,
    grid=(_NP // _NBLK,),
    in_specs=[
        pl.BlockSpec((2, _NBLK, 16), lambda i: (0, i, 0)),
        pl.BlockSpec((2, _NBLK, 16), lambda i: (0, i, 0)),
        pl.BlockSpec((_NBLK,), lambda i: (i,)),
        pl.BlockSpec((_NBLK, _H), lambda i: (i, 0)),
        pl.BlockSpec((_H, _H), lambda i: (0, 0)),
        pl.BlockSpec((_H,), lambda i: (0,)),
        pl.BlockSpec((_H,), lambda i: (0,)),
    ],
    out_specs=[
        pl.BlockSpec((_NBLK, _H), lambda i: (i, 0)),
        pl.BlockSpec((_NBLK, 16), lambda i: (i, 0)),
        pl.BlockSpec((_NBLK, 16), lambda i: (i, 0)),
        pl.BlockSpec((_NBLK,), lambda i: (i,)),
        pl.BlockSpec((_NBLK,), lambda i: (i,)),
        pl.BlockSpec((1, 1), lambda i: (0, 0)),
        pl.BlockSpec((1, 1), lambda i: (0, 0)),
    ],
    out_shape=[_f32((_NP, _H)), _f32((_NP, 16)), _f32((_NP, 16)),
               _f32((_NP,)), _f32((_NP,)), _f32((1, 1)), _f32((1, 1))],
)

_FINAL = pl.pallas_call(
    _k_final,
    grid=(_NP // _NBLK,),
    in_specs=[
        pl.BlockSpec((2, _NBLK, 16), lambda i: (0, i, 0)),
        pl.BlockSpec((2, _NBLK, 16), lambda i: (0, i, 0)),
        pl.BlockSpec((_NBLK,), lambda i: (i,)),
        pl.BlockSpec((_NBLK, _H), lambda i: (i, 0)),
        pl.BlockSpec((_H,), lambda i: (0,)),
    ],
    out_specs=[
        pl.BlockSpec((_NBLK, _H), lambda i: (i, 0)),
        pl.BlockSpec((_NBLK,), lambda i: (i,)),
        pl.BlockSpec((1, 1), lambda i: (0, 0)),
    ],
    out_shape=[_f32((_NP, _H)), _f32((_NP,)), _f32((1, 1))],
)

_EDGE = pl.pallas_call(
    _k_edge,
    grid=(_EP // _EBLK,),
    in_specs=[
        pl.BlockSpec((_EBLK, 9), lambda j: (j, 0)),
        pl.BlockSpec((1, 9, _H), lambda j: (0, 0, 0)),
        pl.BlockSpec((1, _H), lambda j: (0, 0)),
    ],
    out_specs=[
        pl.BlockSpec((1, _EBLK, 16), lambda j: (0, j, 0)),
        pl.BlockSpec((1, _EBLK, 16), lambda j: (0, j, 0)),
        pl.BlockSpec((1, _EBLK), lambda j: (0, j)),
        pl.BlockSpec((1, 1), lambda j: (0, 0)),
    ],
    out_shape=[_f32((1, _EP, 16)), _f32((1, _EP, 16)), _f32((1, _EP)),
               _f32((1, 1))],
)

_MLP = pl.pallas_call(
    _k_mlp,
    grid=(1,),
    in_specs=[
        pl.BlockSpec((2, _BP, _H), lambda i: (0, 0, 0)),
        pl.BlockSpec((2, _BP), lambda i: (0, 0)),
        pl.BlockSpec((_B,), lambda i: (0,)),
        pl.BlockSpec((_B,), lambda i: (0,)),
        pl.BlockSpec((_B,), lambda i: (0,)),
        pl.BlockSpec((_H + 3, _H), lambda i: (0, 0)),
        pl.BlockSpec((_H,), lambda i: (0,)),
        pl.BlockSpec((_H, 16), lambda i: (0, 0)),
        pl.BlockSpec((16,), lambda i: (0,)),
        pl.BlockSpec((16, 1), lambda i: (0, 0)),
        pl.BlockSpec((1,), lambda i: (0,)),
    ],
    out_specs=pl.BlockSpec((_B,), lambda i: (0,)),
    out_shape=_f32((_B,)),
)


def _sck1_body(l, src2, dst2, ea_h, cap_h, hs_h, hd_h, zn1_h,
               w_h, den_h,
               sidxb, didxb, eavb, hsgb, hdgb, wcbb, capv, denom_sh, sem):
    t = lax.axis_index("s")
    c = lax.axis_index("c")
    pltpu.sync_copy(cap_h, capv)
    pltpu.sync_copy(zn1_h, denom_sh.at[pl.ds(t * _NPT, _NPT)])
    plsc.subcore_barrier()
    cap = capv[...]

    @pl.loop(0, _NB1)
    def _(b):
        g0 = t * _CH1 + b * _BC
        e0 = g0 * _EC
        d1 = pltpu.async_copy(src2.at[pl.ds(g0, _BC), :], sidxb, sem)
        d2 = pltpu.async_copy(dst2.at[pl.ds(g0, _BC), :], didxb, sem)
        d3 = pltpu.async_copy(ea_h.at[l, pl.ds(e0, _EB)], eavb, sem)
        d1.wait()
        d2.wait()
        d3.wait()
        gs = []
        for j in range(_BC):
            gs.append(pltpu.async_copy(
                hs_h.at[sidxb.at[j]], hsgb.at[pl.ds(j * _EC, _EC)], sem))
            gs.append(pltpu.async_copy(
                hd_h.at[didxb.at[j]], hdgb.at[pl.ds(j * _EC, _EC)], sem))
        for g in gs:
            g.wait()
        for k in range(_EB // 16):
            sl = pl.ds(k * 16, 16)
            lg = hsgb[sl] + hdgb[sl] + eavb[sl]
            lg = jnp.where(lg > 0, lg, 0.2 * lg)
            wcbb[sl] = jnp.exp(lg - cap)

        own = (b >= c * _NB2) & (b < (c + 1) * _NB2)

        @pl.when(own)
        def _():
            pltpu.sync_copy(wcbb, w_h.at[pl.ds(e0, _EB)])

        ss = []
        for j in range(_BC):
            ss.append(pltpu.async_copy(
                wcbb.at[pl.ds(j * _EC, _EC)], denom_sh.at[didxb.at[j]],
                sem, add=True))
        for s in ss:
            s.wait()

    plsc.subcore_barrier()
    w0 = (2 * t + c) * _NPW
    pltpu.sync_copy(denom_sh.at[pl.ds(w0, _NPW)], den_h.at[pl.ds(w0, _NPW)])


def _sck1(l, src2, dst2, ea, cap16, hs, hd, zn1):
    f = pl.kernel(
        functools.partial(_sck1_body, l),
        out_type=[_f32((_EP,)), _f32((_NP,))],
        mesh=_VMESH,
        compiler_params=_SC_PARAMS,
        scratch_types=[
            pltpu.VMEM((_BC, _EC), jnp.int32),
            pltpu.VMEM((_BC, _EC), jnp.int32),
            pltpu.VMEM((_EB,), jnp.float32),
            pltpu.VMEM((_EB,), jnp.float32),
            pltpu.VMEM((_EB,), jnp.float32),
            pltpu.VMEM((_EB,), jnp.float32),
            pltpu.VMEM((16,), jnp.float32),
            pltpu.VMEM_SHARED((_NP,), jnp.float32),
            pltpu.SemaphoreType.DMA,
        ],
    )
    return f(src2, dst2, ea, cap16, hs, hd, zn1)


def _sck2_body(l, src2, dst2, w_h, hWa_h, hWb_h, eWa_h, eWb_h, zn16_h,
               aggA_h, aggB_h,
               sidx, didx, av, rows, erows, agg_sh, sem):
    t = lax.axis_index("s")
    c = lax.axis_index("c")
    pltpu.sync_copy(zn16_h, agg_sh.at[pl.ds(t * _NPT, _NPT), :])
    plsc.subcore_barrier()

    def half(hW_h, eW_h, agg_out):
        @pl.loop(0, _CH2)
        def _(i):
            gi = t * _CH1 + c * _CH2 + i
            e0 = gi * _EC
            d1 = pltpu.async_copy(src2.at[gi], sidx, sem)
            d2 = pltpu.async_copy(dst2.at[gi], didx, sem)
            d3 = pltpu.async_copy(w_h.at[pl.ds(e0, _EC)],
                                  av.at[pl.ds(0, _EC)], sem)
            d4 = pltpu.async_copy(eW_h.at[l, pl.ds(e0, _EC), :], erows, sem)
            d1.wait()
            g = pltpu.async_copy(hW_h.at[sidx], rows, sem)
            d2.wait()
            d3.wait()
            d4.wait()
            g.wait()

            @pl.loop(0, _EC)
            def _(r):
                a = av[pl.ds(r, 16)][0]
                rows[r] = (rows[r] + erows[r]) * a

            pltpu.sync_copy(rows, agg_sh.at[didx], add=True)

        plsc.subcore_barrier()
        pltpu.sync_copy(agg_sh.at[pl.ds(t * _NPT, _NPT), :],
                        agg_out.at[c, pl.ds(t * _NPT, _NPT), :])

    half(hWa_h, eWa_h, aggA_h)
    pltpu.sync_copy(zn16_h, agg_sh.at[pl.ds(t * _NPT, _NPT), :])
    plsc.subcore_barrier()
    half(hWb_h, eWb_h, aggB_h)


def _sck2(l, src2, dst2, w, hWa, hWb, eWa, eWb, zn16):
    f = pl.kernel(
        functools.partial(_sck2_body, l),
        out_type=[_f32((2, _NP, 16)), _f32((2, _NP, 16))],
        mesh=_VMESH,
        compiler_params=_SC_PARAMS,
        scratch_types=[
            pltpu.VMEM((_EC,), jnp.int32),
            pltpu.VMEM((_EC,), jnp.int32),
            pltpu.VMEM((_EC + 16,), jnp.float32),
            pltpu.VMEM((_EC, 16), jnp.float32),
            pltpu.VMEM((_EC, 16), jnp.float32),
            pltpu.VMEM_SHARED((_NP, 16), jnp.float32),
            pltpu.SemaphoreType.DMA,
        ],
    )
    return f(src2, dst2, w, hWa, hWb, eWa, eWb, zn16)


def _sc_pool_body(batch2, s2, h3, cap_h, zb1_h, zb16_h,
                  pooled_h, wd_h,
                  bidxb, svb, wsbb, hrowb, capv, wd_sh, pool_sh, sem):
    t = lax.axis_index("s")
    c = lax.axis_index("c")
    pltpu.sync_copy(cap_h, capv)
    pltpu.sync_copy(zb1_h, wd_sh.at[pl.ds(t * _BPT, _BPT)])
    pltpu.sync_copy(zb16_h, pool_sh.at[pl.ds(t * _BPT, _BPT), :, :])
    plsc.subcore_barrier()
    cap = capv[...]

    @pl.loop(0, 7)
    def _(b):
        g0 = t * _PCH1 + b * 7
        n0 = g0 * _EC
        d1 = pltpu.async_copy(batch2.at[pl.ds(g0, 7), :], bidxb, sem)
        d2 = pltpu.async_copy(s2.at[pl.ds(g0, 7), :], svb, sem)
        d3 = pltpu.async_copy(h3.at[pl.ds(n0, 7 * _EC), :, :], hrowb, sem)
        d1.wait()
        d2.wait()
        d3.wait()
        for j in range(7):
            for k in range(8):
                wsbb[pl.ds((j * 8 + k) * 16, 16)] = jnp.exp(
                    svb[j, pl.ds(k * 16, 16)] - cap)
        ws_sc = []
        for j in range(7):
            ws_sc.append(pltpu.async_copy(
                wsbb.at[pl.ds(j * _EC, _EC)], wd_sh.at[bidxb.at[j]],
                sem, add=True))

        @pl.loop(0, 7 * _EC)
        def _(r):
            a = wsbb[pl.ds(r, 16)][0]
            hrowb[r, 0] = hrowb[r, 0] * a
            hrowb[r, 1] = hrowb[r, 1] * a

        for s in ws_sc:
            s.wait()
        ps = []
        for j in range(7):
            ps.append(pltpu.async_copy(
                hrowb.at[pl.ds(j * _EC, _EC), :, :],
                pool_sh.at[bidxb.at[j]], sem, add=True))
        for s in ps:
            s.wait()

    plsc.subcore_barrier()
    pltpu.sync_copy(pool_sh.at[pl.ds(t * _BPT, _BPT), :, :],
                    pooled_h.at[c, pl.ds(t * _BPT, _BPT), :, :])
    pltpu.sync_copy(wd_sh.at[pl.ds(t * _BPT, _BPT)],
                    wd_h.at[c, pl.ds(t * _BPT, _BPT)])


def _sc_pool(batch2, s2, h3, cap16, zb1, zb16):
    f = pl.kernel(
        _sc_pool_body,
        out_type=[_f32((2, _BP, 2, 16)), _f32((2, _BP))],
        mesh=_VMESH,
        compiler_params=_SC_PARAMS,
        scratch_types=[
            pltpu.VMEM((7, _EC), jnp.int32),
            pltpu.VMEM((7, _EC), jnp.float32),
            pltpu.VMEM((7 * _EC + 16,), jnp.float32),
            pltpu.VMEM((7 * _EC, 2, 16), jnp.float32),
            pltpu.VMEM((16,), jnp.float32),
            pltpu.VMEM_SHARED((_BP,), jnp.float32),
            pltpu.VMEM_SHARED((_BP, 2, 16), jnp.float32),
            pltpu.SemaphoreType.DMA,
        ],
    )
    return f(batch2, s2, h3, cap16, zb1, zb16)


def kernel(x, temperature, edge_index, edge_attr, numHDonors, numHAcceptors,
           batch, W0, b0, Wl, Wel, a_s, a_d, a_e, w_pool, W1, b1, W2, b2,
           W3, b3):
    x_p = jnp.zeros((_NP, 24), jnp.float32).at[:_N].set(x)
    src = jnp.zeros((_EP,), jnp.int32).at[:_E].set(edge_index[0])
    dst = jnp.full((_EP,), _N, jnp.int32).at[:_E].set(edge_index[1])
    eattr_p = jnp.zeros((_EP, 9), jnp.float32).at[:_E].set(edge_attr)
    batch_p = jnp.full((_NP,), _B, jnp.int32).at[:_N].set(batch)
    src2 = src.reshape(_ECH, _EC)
    dst2 = dst.reshape(_ECH, _EC)
    batch2 = batch_p.reshape(_NCH, _EC)
    zn16 = jnp.zeros((_NPT, 16), jnp.float32)
    zn1 = jnp.zeros((_NPT,), jnp.float32)
    zb1 = jnp.zeros((_BPT,), jnp.float32)
    zb16 = jnp.zeros((_BPT, 2, 16), jnp.float32)

    h, hWa, hWb, hs, hd, mhs, mhd = _NODE0(x_p, W0, b0, Wl[0], a_s[0], a_d[0])
    for l in range(3):
        eWa1, eWb1, ea1, mea = _EDGE(eattr_p, Wel[l:l + 1], a_e[l:l + 1])
        cap = jnp.maximum(mhs[0, 0] + mhd[0, 0] + mea[0, 0], 0.0)
        cap16 = jnp.full((16,), cap, jnp.float32)
        w, den = _sck1(0, src2, dst2, ea1, cap16, hs, hd, zn1)
        aggA, aggB = _sck2(0, src2, dst2, w, hWa, hWb, eWa1, eWb1, zn16)
        if l < 2:
            h, hWa, hWb, hs, hd, mhs, mhd = _NODE(
                aggA, aggB, den, h, Wl[l + 1], a_s[l + 1], a_d[l + 1])
        else:
            h, s, ms = _FINAL(aggA, aggB, den, h, w_pool)
    cap16s = jnp.full((16,), ms[0, 0], jnp.float32)
    pooled4, wd2 = _sc_pool(batch2, s.reshape(_NCH, _EC),
                            h.reshape(_NP, 2, 16), cap16s, zb1, zb16)
    return _MLP(pooled4.reshape(2, _BP, _H), wd2, temperature,
                numHDonors.astype(jnp.float32),
                numHAcceptors.astype(jnp.float32), W1, b1, W2, b2, W3, b3)
